# baseline (speedup 1.0000x reference)
import jax
import jax.numpy as jnp
from jax.experimental import pallas as pl
from jax.experimental.pallas import tpu as pltpu
from jax.experimental.pallas import tpu_sc as plsc

_BLK_ROWS = 16
_LANES = 16


def kernel(x, pos_table):
    batch, seq_len, d_model = x.shape
    total_rows = batch * seq_len
    x2 = x.reshape(total_rows, d_model)
    pos = pos_table[:seq_len]
    n_blocks = total_rows // _BLK_ROWS
    n_pos_blocks = seq_len // _BLK_ROWS

    mesh = plsc.VectorSubcoreMesh(core_axis_name="core", subcore_axis_name="subcore")

    @pl.kernel(
        out_type=jax.ShapeDtypeStruct((total_rows, d_model), x.dtype),
        mesh=mesh,
    )
    def sc_add(x_hbm, pos_hbm, o_hbm):
        def body(x_vmem, pos_vmem, o_vmem):
            @pl.loop(0, _BLK_ROWS)
            def _(r):
                @pl.loop(0, d_model, step=_LANES)
                def _(c):
                    slc = (pl.ds(r, 1), pl.ds(c, _LANES))
                    o_vmem.at[*slc][...] = (
                        x_vmem.at[*slc][...] + pos_vmem.at[*slc][...]
                    )

        pltpu.emit_pipeline(
            body,
            grid=(n_blocks,),
            in_specs=[
                pl.BlockSpec((_BLK_ROWS, d_model), index_map=lambda i: (i, 0)),
                pl.BlockSpec(
                    (_BLK_ROWS, d_model),
                    index_map=lambda i: (i % n_pos_blocks, 0),
                ),
            ],
            out_specs=[
                pl.BlockSpec((_BLK_ROWS, d_model), index_map=lambda i: (i, 0))
            ],
            core_axis_name=("core", "subcore"),
            dimension_semantics=(pltpu.PARALLEL,),
        )(x_hbm, pos_hbm, o_hbm)

    return sc_add(x2, pos).reshape(batch, seq_len, d_model)
